# baseline (speedup 1.0000x reference)
import functools

import jax
import jax.numpy as jnp
from jax import lax
from jax.experimental import pallas as pl
from jax.experimental.pallas import tpu as pltpu
from jax.experimental.pallas import tpu_sc as plsc

N = 10000
D = 128
E = 320000
N_CLS = 10
NC, NS = 2, 16
NW = NC * NS
EW = E // NW
CH = 40
NCHUNK = EW // CH
RT_A = 624
RT_LAST = N - (NS - 1) * RT_A
DEGW = 16

_MESH = plsc.VectorSubcoreMesh(
    core_axis_name="c", subcore_axis_name="s", num_cores=NC, num_subcores=NS)


def _per_tile_rows(s, fn):
    r0 = s * RT_A

    @pl.when(s < NS - 1)
    def _():
        fn(r0, RT_A)

    @pl.when(s == NS - 1)
    def _():
        fn(r0, RT_LAST)


def _deg_body(col_hbm, ones_hbm, zeros_hbm, out_hbm, idx_v, ones_v, acc,
              sm0, sm1):
    c = lax.axis_index("c")
    s = lax.axis_index("s")
    wid = s * NC + c
    pltpu.sync_copy(ones_hbm, ones_v)
    pltpu.sync_copy(col_hbm.at[wid], idx_v)
    _per_tile_rows(s, lambda r0, nr: pltpu.sync_copy(
        zeros_hbm.at[pl.ds(r0, nr)], acc.at[pl.ds(r0, nr)]))
    plsc.subcore_barrier()

    def _scat(j, b, ssems):
        pltpu.async_copy(ones_v, acc.at[idx_v.at[j]], ssems[b], add=True)

    def _sdrain(b, ssems):
        pltpu.make_async_copy(ones_v, acc.at[idx_v.at[0]], ssems[b]).wait()

    def step(j, carry):
        def one(b):
            @pl.when(j >= 2)
            def _():
                _sdrain(b, (sm0, sm1))

            _scat(j, b, (sm0, sm1))

        @pl.when(j % 2 == 0)
        def _():
            one(0)

        @pl.when(j % 2 == 1)
        def _():
            one(1)

        return carry

    lax.fori_loop(0, NCHUNK, step, 0)
    _sdrain(0, (sm0, sm1))
    _sdrain(1, (sm0, sm1))
    plsc.subcore_barrier()
    _per_tile_rows(s, lambda r0, nr: pltpu.sync_copy(
        acc.at[pl.ds(r0, nr)], out_hbm.at[c, pl.ds(r0, nr)]))


_deg_call = pl.kernel(
    _deg_body,
    out_type=jax.ShapeDtypeStruct((NC, N, D), jnp.float32),
    mesh=_MESH,
    scratch_types=[
        pltpu.VMEM((NCHUNK, CH), jnp.int32),
        pltpu.VMEM((CH, D), jnp.float32),
        pltpu.VMEM_SHARED((N, D), jnp.float32),
        pltpu.SemaphoreType.DMA,
        pltpu.SemaphoreType.DMA,
    ],
)


NB = 4


def _agg_body(hp_hbm, rc_hbm, zeros_hbm, out_hbm, rc_v, rows_v, acc,
              is0, is1, is2, is3, gs0, gs1, gs2, gs3, ss0, ss1, ss2, ss3):
    c = lax.axis_index("c")
    s = lax.axis_index("s")
    wid = s * NC + c
    isems = (is0, is1, is2, is3)
    gsems = (gs0, gs1, gs2, gs3)
    ssems = (ss0, ss1, ss2, ss3)

    def _init(r0, nr):
        @pl.when(c == 0)
        def _():
            pltpu.sync_copy(hp_hbm.at[pl.ds(r0, nr)], acc.at[pl.ds(r0, nr)])

        @pl.when(c != 0)
        def _():
            pltpu.sync_copy(zeros_hbm.at[pl.ds(r0, nr)], acc.at[pl.ds(r0, nr)])

    _per_tile_rows(s, _init)

    def fire_idx(j, b):
        pltpu.async_copy(rc_hbm.at[wid, j], rc_v.at[b], isems[b])

    def wait_idx(b):
        pltpu.make_async_copy(rc_hbm.at[wid, 0], rc_v.at[b], isems[b]).wait()

    def fire_gather(j, b):
        pltpu.async_copy(hp_hbm.at[rc_v.at[b, 0]], rows_v.at[b], gsems[b])

    def wait_gather(b):
        pltpu.make_async_copy(
            hp_hbm.at[rc_v.at[b, 0]], rows_v.at[b], gsems[b]).wait()

    def fire_scat(j, b):
        pltpu.async_copy(
            rows_v.at[b], acc.at[rc_v.at[b, 1]], ssems[b], add=True)

    def wait_scat(b):
        pltpu.make_async_copy(
            rows_v.at[b], acc.at[rc_v.at[b, 1]], ssems[b]).wait()

    fire_idx(0, 0)
    fire_idx(1, 1)
    fire_idx(2, 2)
    wait_idx(0)
    fire_gather(0, 0)
    wait_idx(1)
    fire_gather(1, 1)
    plsc.subcore_barrier()

    def body(j, carry):
        def stepb(b):
            b2 = (b + 2) % NB
            b3 = (b + 3) % NB

            @pl.when(j + 3 < NCHUNK)
            def _():
                @pl.when(j >= 1)
                def _():
                    wait_scat(b3)

                fire_idx(j + 3, b3)

            @pl.when(j + 2 < NCHUNK)
            def _():
                wait_idx(b2)
                fire_gather(j + 2, b2)

            wait_gather(b)
            fire_scat(j, b)

        for bb in range(NB):
            @pl.when(j % NB == bb)
            def _(bb=bb):
                stepb(bb)

        return carry

    lax.fori_loop(0, NCHUNK, body, 0)
    for b in range(NB):
        wait_scat(b)
    plsc.subcore_barrier()
    _per_tile_rows(s, lambda r0, nr: pltpu.sync_copy(
        acc.at[pl.ds(r0, nr)], out_hbm.at[c, pl.ds(r0, nr)]))


_agg_call = pl.kernel(
    _agg_body,
    out_type=jax.ShapeDtypeStruct((NC, N, D), jnp.float32),
    mesh=_MESH,
    scratch_types=[
        pltpu.VMEM((NB, 2, CH), jnp.int32),
        pltpu.VMEM((NB, CH, D), jnp.float32),
        pltpu.VMEM_SHARED((N, D), jnp.float32),
    ] + [pltpu.SemaphoreType.DMA] * (3 * NB),
)


BN = 2000


def _dinv(degp_ref):
    deg = degp_ref[0, :, 0] + degp_ref[1, :, 0] + 1.0
    return lax.rsqrt(deg)[:, None]


def _tc_first_body(x_ref, w_ref, degp_ref, out_ref):
    dinv = _dinv(degp_ref)
    t = jnp.dot(x_ref[...], w_ref[...], preferred_element_type=jnp.float32)
    out_ref[...] = dinv * t


def _tc_mid_body(aggp_ref, degp_ref, b_ref, w_ref, out_ref):
    dinv = _dinv(degp_ref)
    h = jax.nn.relu(dinv * (aggp_ref[0] + aggp_ref[1]) + b_ref[...])
    t = jnp.dot(h, w_ref[...], preferred_element_type=jnp.float32)
    out_ref[...] = dinv * t


def _tc_final_body(aggp_ref, degp_ref, b3_ref, wf1_ref, bf1_ref, wf2_ref,
                   bf2_ref, out_ref):
    dinv = _dinv(degp_ref)
    h = jax.nn.relu(dinv * (aggp_ref[0] + aggp_ref[1]) + b3_ref[...])
    h = jax.nn.relu(
        jnp.dot(h, wf1_ref[...], preferred_element_type=jnp.float32)
        + bf1_ref[...])
    out_ref[...] = (
        jnp.dot(h, wf2_ref[...], preferred_element_type=jnp.float32)
        + bf2_ref[...])


def _row_block(i):
    return (i, 0)


_spec_x = pl.BlockSpec((BN, D), _row_block)
_spec_aggp = pl.BlockSpec((NC, BN, D), lambda i: (0, i, 0))
_spec_degp = pl.BlockSpec((NC, BN, D), lambda i: (0, i, 0))
_spec_w = pl.BlockSpec((D, D), lambda i: (0, 0))
_spec_b = pl.BlockSpec((1, D), lambda i: (0, 0))
_spec_wf2 = pl.BlockSpec((D, N_CLS), lambda i: (0, 0))
_spec_bf2 = pl.BlockSpec((1, N_CLS), lambda i: (0, 0))

_GRID = N // BN

_tc_first = pl.pallas_call(
    _tc_first_body,
    grid=(_GRID,),
    in_specs=[_spec_x, _spec_w, _spec_degp],
    out_specs=_spec_x,
    out_shape=jax.ShapeDtypeStruct((N, D), jnp.float32),
)

_tc_mid = pl.pallas_call(
    _tc_mid_body,
    grid=(_GRID,),
    in_specs=[_spec_aggp, _spec_degp, _spec_b, _spec_w],
    out_specs=_spec_x,
    out_shape=jax.ShapeDtypeStruct((N, D), jnp.float32),
)

_tc_final = pl.pallas_call(
    _tc_final_body,
    grid=(_GRID,),
    in_specs=[_spec_aggp, _spec_degp, _spec_b, _spec_w, _spec_b,
              _spec_wf2, _spec_bf2],
    out_specs=pl.BlockSpec((BN, N_CLS), _row_block),
    out_shape=jax.ShapeDtypeStruct((N, N_CLS), jnp.float32),
)


def kernel(x, edge_index, W1, b1, W2, b2, W3, b3, Wf1, bf1, Wf2, bf2):
    row3 = edge_index[0].reshape(NW, NCHUNK, CH)
    col3 = edge_index[1].reshape(NW, NCHUNK, CH)
    rc = jnp.stack([row3, col3], axis=2)
    zeros = jnp.zeros((N, D), jnp.float32)
    ones_ch = jnp.ones((CH, D), jnp.float32)

    degp = _deg_call(col3, ones_ch, zeros)

    t1 = _tc_first(x, W1, degp)
    a1 = _agg_call(t1, rc, zeros)
    t2 = _tc_mid(a1, degp, b1.reshape(1, D), W2)
    a2 = _agg_call(t2, rc, zeros)
    t3 = _tc_mid(a2, degp, b2.reshape(1, D), W3)
    a3 = _agg_call(t3, rc, zeros)
    out = _tc_final(a3, degp, b3.reshape(1, D), Wf1, bf1.reshape(1, D),
                    Wf2, bf2.reshape(1, N_CLS))
    return out


# speedup vs baseline: 1.2218x; 1.2218x over previous
import jax
import jax.numpy as jnp
from jax import lax
from jax.experimental import pallas as pl
from jax.experimental.pallas import tpu as pltpu
from jax.experimental.pallas import tpu_sc as plsc

N = 10000
D = 128
E = 320000
N_CLS = 10
NC, NS = 2, 16
NW = NC * NS
EW = E // NW
CH = 80
NCHUNK = EW // CH
RT_A = 624
RT_LAST = N - (NS - 1) * RT_A

_MESH = plsc.VectorSubcoreMesh(
    core_axis_name="c", subcore_axis_name="s", num_cores=NC, num_subcores=NS)


def _per_tile_rows(s, fn):
    r0 = s * RT_A

    @pl.when(s < NS - 1)
    def _():
        fn(r0, RT_A)

    @pl.when(s == NS - 1)
    def _():
        fn(r0, RT_LAST)


def _deg_body(col_hbm, ones_hbm, zeros_hbm, out_hbm, idx_v, ones_v, acc,
              sm0, sm1):
    c = lax.axis_index("c")
    s = lax.axis_index("s")
    wid = s * NC + c
    pltpu.sync_copy(ones_hbm, ones_v)
    pltpu.sync_copy(col_hbm.at[wid], idx_v)
    _per_tile_rows(s, lambda r0, nr: pltpu.sync_copy(
        zeros_hbm.at[pl.ds(r0, nr)], acc.at[pl.ds(r0, nr)]))
    plsc.subcore_barrier()

    def _scat(j, b, ssems):
        pltpu.async_copy(ones_v, acc.at[idx_v.at[j]], ssems[b], add=True)

    def _sdrain(b, ssems):
        pltpu.make_async_copy(ones_v, acc.at[idx_v.at[0]], ssems[b]).wait()

    def step(j, carry):
        def one(b):
            @pl.when(j >= 2)
            def _():
                _sdrain(b, (sm0, sm1))

            _scat(j, b, (sm0, sm1))

        @pl.when(j % 2 == 0)
        def _():
            one(0)

        @pl.when(j % 2 == 1)
        def _():
            one(1)

        return carry

    lax.fori_loop(0, NCHUNK, step, 0)
    _sdrain(0, (sm0, sm1))
    _sdrain(1, (sm0, sm1))
    plsc.subcore_barrier()
    _per_tile_rows(s, lambda r0, nr: pltpu.sync_copy(
        acc.at[pl.ds(r0, nr)], out_hbm.at[c, pl.ds(r0, nr)]))


_deg_call = pl.kernel(
    _deg_body,
    out_type=jax.ShapeDtypeStruct((NC, N, D), jnp.float32),
    mesh=_MESH,
    scratch_types=[
        pltpu.VMEM((NCHUNK, CH), jnp.int32),
        pltpu.VMEM((CH, D), jnp.float32),
        pltpu.VMEM_SHARED((N, D), jnp.float32),
        pltpu.SemaphoreType.DMA,
        pltpu.SemaphoreType.DMA,
    ],
)


NB = 4


def _agg_body(hp_hbm, rc_hbm, zeros_hbm, out_hbm, rc_v, rows_v, acc,
              is0, is1, is2, is3, gs0, gs1, gs2, gs3, ss0, ss1, ss2, ss3):
    c = lax.axis_index("c")
    s = lax.axis_index("s")
    wid = s * NC + c
    isems = (is0, is1, is2, is3)
    gsems = (gs0, gs1, gs2, gs3)
    ssems = (ss0, ss1, ss2, ss3)

    def _init(r0, nr):
        @pl.when(c == 0)
        def _():
            pltpu.sync_copy(hp_hbm.at[pl.ds(r0, nr)], acc.at[pl.ds(r0, nr)])

        @pl.when(c != 0)
        def _():
            pltpu.sync_copy(zeros_hbm.at[pl.ds(r0, nr)], acc.at[pl.ds(r0, nr)])

    _per_tile_rows(s, _init)

    def fire_idx(j, b):
        pltpu.async_copy(rc_hbm.at[wid, j], rc_v.at[b], isems[b])

    def wait_idx(b):
        pltpu.make_async_copy(rc_hbm.at[wid, 0], rc_v.at[b], isems[b]).wait()

    def fire_gather(j, b):
        pltpu.async_copy(hp_hbm.at[rc_v.at[b, 0]], rows_v.at[b], gsems[b])

    def wait_gather(b):
        pltpu.make_async_copy(
            hp_hbm.at[rc_v.at[b, 0]], rows_v.at[b], gsems[b]).wait()

    def fire_scat(j, b):
        pltpu.async_copy(
            rows_v.at[b], acc.at[rc_v.at[b, 1]], ssems[b], add=True)

    def wait_scat(b):
        pltpu.make_async_copy(
            rows_v.at[b], acc.at[rc_v.at[b, 1]], ssems[b]).wait()

    fire_idx(0, 0)
    fire_idx(1, 1)
    fire_idx(2, 2)
    wait_idx(0)
    fire_gather(0, 0)
    wait_idx(1)
    fire_gather(1, 1)
    plsc.subcore_barrier()

    def body(j, carry):
        def stepb(b):
            b2 = (b + 2) % NB
            b3 = (b + 3) % NB

            @pl.when(j + 3 < NCHUNK)
            def _():
                @pl.when(j >= 1)
                def _():
                    wait_scat(b3)

                fire_idx(j + 3, b3)

            @pl.when(j + 2 < NCHUNK)
            def _():
                wait_idx(b2)
                fire_gather(j + 2, b2)

            wait_gather(b)
            fire_scat(j, b)

        for bb in range(NB):
            @pl.when(j % NB == bb)
            def _(bb=bb):
                stepb(bb)

        return carry

    lax.fori_loop(0, NCHUNK, body, 0)
    for b in range(NB):
        wait_scat(b)
    plsc.subcore_barrier()
    _per_tile_rows(s, lambda r0, nr: pltpu.sync_copy(
        acc.at[pl.ds(r0, nr)], out_hbm.at[c, pl.ds(r0, nr)]))


_agg_call = pl.kernel(
    _agg_body,
    out_type=jax.ShapeDtypeStruct((NC, N, D), jnp.float32),
    mesh=_MESH,
    scratch_types=[
        pltpu.VMEM((NB, 2, CH), jnp.int32),
        pltpu.VMEM((NB, CH, D), jnp.float32),
        pltpu.VMEM_SHARED((N, D), jnp.float32),
    ] + [pltpu.SemaphoreType.DMA] * (3 * NB),
)


BN = 2000


def _dinv(degp_ref):
    deg = degp_ref[0, :, 0] + degp_ref[1, :, 0] + 1.0
    return lax.rsqrt(deg)[:, None]


def _tc_first_body(x_ref, w_ref, degp_ref, out_ref):
    dinv = _dinv(degp_ref)
    t = jnp.dot(x_ref[...], w_ref[...], preferred_element_type=jnp.float32)
    out_ref[...] = dinv * t


def _tc_mid_body(aggp_ref, degp_ref, b_ref, w_ref, out_ref):
    dinv = _dinv(degp_ref)
    h = jax.nn.relu(dinv * (aggp_ref[0] + aggp_ref[1]) + b_ref[...])
    t = jnp.dot(h, w_ref[...], preferred_element_type=jnp.float32)
    out_ref[...] = dinv * t


def _tc_final_body(aggp_ref, degp_ref, b3_ref, wf1_ref, bf1_ref, wf2_ref,
                   bf2_ref, out_ref):
    dinv = _dinv(degp_ref)
    h = jax.nn.relu(dinv * (aggp_ref[0] + aggp_ref[1]) + b3_ref[...])
    h = jax.nn.relu(
        jnp.dot(h, wf1_ref[...], preferred_element_type=jnp.float32)
        + bf1_ref[...])
    out_ref[...] = (
        jnp.dot(h, wf2_ref[...], preferred_element_type=jnp.float32)
        + bf2_ref[...])


def _row_block(i):
    return (i, 0)


_spec_x = pl.BlockSpec((BN, D), _row_block)
_spec_aggp = pl.BlockSpec((NC, BN, D), lambda i: (0, i, 0))
_spec_degp = pl.BlockSpec((NC, BN, D), lambda i: (0, i, 0))
_spec_w = pl.BlockSpec((D, D), lambda i: (0, 0))
_spec_b = pl.BlockSpec((1, D), lambda i: (0, 0))
_spec_wf2 = pl.BlockSpec((D, N_CLS), lambda i: (0, 0))
_spec_bf2 = pl.BlockSpec((1, N_CLS), lambda i: (0, 0))

_GRID = N // BN

_tc_first = pl.pallas_call(
    _tc_first_body,
    grid=(_GRID,),
    in_specs=[_spec_x, _spec_w, _spec_degp],
    out_specs=_spec_x,
    out_shape=jax.ShapeDtypeStruct((N, D), jnp.float32),
)

_tc_mid = pl.pallas_call(
    _tc_mid_body,
    grid=(_GRID,),
    in_specs=[_spec_aggp, _spec_degp, _spec_b, _spec_w],
    out_specs=_spec_x,
    out_shape=jax.ShapeDtypeStruct((N, D), jnp.float32),
)

_tc_final = pl.pallas_call(
    _tc_final_body,
    grid=(_GRID,),
    in_specs=[_spec_aggp, _spec_degp, _spec_b, _spec_w, _spec_b,
              _spec_wf2, _spec_bf2],
    out_specs=pl.BlockSpec((BN, N_CLS), _row_block),
    out_shape=jax.ShapeDtypeStruct((N, N_CLS), jnp.float32),
)


def kernel(x, edge_index, W1, b1, W2, b2, W3, b3, Wf1, bf1, Wf2, bf2):
    row3 = edge_index[0].reshape(NW, NCHUNK, CH)
    col3 = edge_index[1].reshape(NW, NCHUNK, CH)
    rc = jnp.stack([row3, col3], axis=2)
    zeros = jnp.zeros((N, D), jnp.float32)
    ones_ch = jnp.ones((CH, D), jnp.float32)

    degp = _deg_call(col3, ones_ch, zeros)

    t1 = _tc_first(x, W1, degp)
    a1 = _agg_call(t1, rc, zeros)
    t2 = _tc_mid(a1, degp, b1.reshape(1, D), W2)
    a2 = _agg_call(t2, rc, zeros)
    t3 = _tc_mid(a2, degp, b2.reshape(1, D), W3)
    a3 = _agg_call(t3, rc, zeros)
    out = _tc_final(a3, degp, b3.reshape(1, D), Wf1, bf1.reshape(1, D),
                    Wf2, bf2.reshape(1, N_CLS))
    return out


# speedup vs baseline: 1.2254x; 1.0030x over previous
import jax
import jax.numpy as jnp
from jax import lax
from jax.experimental import pallas as pl
from jax.experimental.pallas import tpu as pltpu
from jax.experimental.pallas import tpu_sc as plsc

N = 10000
D = 128
E = 320000
N_CLS = 10
NC, NS = 2, 16
NW = NC * NS
EW = E // NW
CH = 80
NCHUNK = EW // CH
RT_A = 624
RT_LAST = N - (NS - 1) * RT_A

_MESH = plsc.VectorSubcoreMesh(
    core_axis_name="c", subcore_axis_name="s", num_cores=NC, num_subcores=NS)


def _per_tile_rows(s, fn):
    r0 = s * RT_A

    @pl.when(s < NS - 1)
    def _():
        fn(r0, RT_A)

    @pl.when(s == NS - 1)
    def _():
        fn(r0, RT_LAST)


def _deg_body(col_hbm, ones_hbm, zeros_hbm, out_hbm, idx_v, ones_v, acc,
              sm0, sm1):
    c = lax.axis_index("c")
    s = lax.axis_index("s")
    wid = s * NC + c
    pltpu.sync_copy(ones_hbm, ones_v)
    pltpu.sync_copy(col_hbm.at[wid], idx_v)
    _per_tile_rows(s, lambda r0, nr: pltpu.sync_copy(
        zeros_hbm.at[pl.ds(r0, nr)], acc.at[pl.ds(r0, nr)]))
    plsc.subcore_barrier()

    def _scat(j, b, ssems):
        pltpu.async_copy(ones_v, acc.at[idx_v.at[j]], ssems[b], add=True)

    def _sdrain(b, ssems):
        pltpu.make_async_copy(ones_v, acc.at[idx_v.at[0]], ssems[b]).wait()

    def step(j, carry):
        def one(b):
            @pl.when(j >= 2)
            def _():
                _sdrain(b, (sm0, sm1))

            _scat(j, b, (sm0, sm1))

        @pl.when(j % 2 == 0)
        def _():
            one(0)

        @pl.when(j % 2 == 1)
        def _():
            one(1)

        return carry

    lax.fori_loop(0, NCHUNK, step, 0)
    _sdrain(0, (sm0, sm1))
    _sdrain(1, (sm0, sm1))
    plsc.subcore_barrier()
    _per_tile_rows(s, lambda r0, nr: pltpu.sync_copy(
        acc.at[pl.ds(r0, nr)], out_hbm.at[c, pl.ds(r0, nr)]))


_deg_call = pl.kernel(
    _deg_body,
    out_type=jax.ShapeDtypeStruct((NC, N, D), jnp.float32),
    mesh=_MESH,
    scratch_types=[
        pltpu.VMEM((NCHUNK, CH), jnp.int32),
        pltpu.VMEM((CH, D), jnp.float32),
        pltpu.VMEM_SHARED((N, D), jnp.float32),
        pltpu.SemaphoreType.DMA,
        pltpu.SemaphoreType.DMA,
    ],
)


NB = 4


def _agg_body(hp_hbm, rc_hbm, zeros_hbm, out_hbm, rc_v, rows_v, acc,
              is0, is1, is2, is3, gs0, gs1, gs2, gs3, ss0, ss1, ss2, ss3):
    c = lax.axis_index("c")
    s = lax.axis_index("s")
    wid = s * NC + c
    isems = (is0, is1, is2, is3)
    gsems = (gs0, gs1, gs2, gs3)
    ssems = (ss0, ss1, ss2, ss3)

    def _init(r0, nr):
        @pl.when(c == 0)
        def _():
            pltpu.sync_copy(hp_hbm.at[pl.ds(r0, nr)], acc.at[pl.ds(r0, nr)])

        @pl.when(c != 0)
        def _():
            pltpu.sync_copy(zeros_hbm.at[pl.ds(r0, nr)], acc.at[pl.ds(r0, nr)])

    def fire_idx(j, b):
        pltpu.async_copy(rc_hbm.at[wid, j], rc_v.at[b], isems[b])

    def wait_idx(b):
        pltpu.make_async_copy(rc_hbm.at[wid, 0], rc_v.at[b], isems[b]).wait()

    def fire_gather(j, b):
        pltpu.async_copy(hp_hbm.at[rc_v.at[b, 0]], rows_v.at[b], gsems[b])

    def wait_gather(b):
        pltpu.make_async_copy(
            hp_hbm.at[rc_v.at[b, 0]], rows_v.at[b], gsems[b]).wait()

    def fire_scat(j, b):
        pltpu.async_copy(
            rows_v.at[b], acc.at[rc_v.at[b, 1]], ssems[b], add=True)

    def wait_scat(b):
        pltpu.make_async_copy(
            rows_v.at[b], acc.at[rc_v.at[b, 1]], ssems[b]).wait()

    fire_idx(0, 0)
    fire_idx(1, 1)
    fire_idx(2, 2)
    _per_tile_rows(s, _init)
    wait_idx(0)
    fire_gather(0, 0)
    wait_idx(1)
    fire_gather(1, 1)
    plsc.subcore_barrier()

    def body(j, carry):
        def stepb(b):
            b2 = (b + 2) % NB
            b3 = (b + 3) % NB

            @pl.when(j + 3 < NCHUNK)
            def _():
                @pl.when(j >= 1)
                def _():
                    wait_scat(b3)

                fire_idx(j + 3, b3)

            @pl.when(j + 2 < NCHUNK)
            def _():
                wait_idx(b2)
                fire_gather(j + 2, b2)

            wait_gather(b)
            fire_scat(j, b)

        for bb in range(NB):
            @pl.when(j % NB == bb)
            def _(bb=bb):
                stepb(bb)

        return carry

    lax.fori_loop(0, NCHUNK, body, 0)
    for b in range(NB):
        wait_scat(b)
    plsc.subcore_barrier()
    _per_tile_rows(s, lambda r0, nr: pltpu.sync_copy(
        acc.at[pl.ds(r0, nr)], out_hbm.at[c, pl.ds(r0, nr)]))


_agg_call = pl.kernel(
    _agg_body,
    out_type=jax.ShapeDtypeStruct((NC, N, D), jnp.float32),
    mesh=_MESH,
    scratch_types=[
        pltpu.VMEM((NB, 2, CH), jnp.int32),
        pltpu.VMEM((NB, CH, D), jnp.float32),
        pltpu.VMEM_SHARED((N, D), jnp.float32),
    ] + [pltpu.SemaphoreType.DMA] * (3 * NB),
)


BN = 2000


def _dinv(degp_ref):
    deg = degp_ref[0, :, 0] + degp_ref[1, :, 0] + 1.0
    return lax.rsqrt(deg)[:, None]


def _tc_first_body(x_ref, w_ref, degp_ref, out_ref):
    dinv = _dinv(degp_ref)
    t = jnp.dot(x_ref[...], w_ref[...], preferred_element_type=jnp.float32)
    out_ref[...] = dinv * t


def _tc_mid_body(aggp_ref, degp_ref, b_ref, w_ref, out_ref):
    dinv = _dinv(degp_ref)
    h = jax.nn.relu(dinv * (aggp_ref[0] + aggp_ref[1]) + b_ref[...])
    t = jnp.dot(h, w_ref[...], preferred_element_type=jnp.float32)
    out_ref[...] = dinv * t


def _tc_final_body(aggp_ref, degp_ref, b3_ref, wf1_ref, bf1_ref, wf2_ref,
                   bf2_ref, out_ref):
    dinv = _dinv(degp_ref)
    h = jax.nn.relu(dinv * (aggp_ref[0] + aggp_ref[1]) + b3_ref[...])
    h = jax.nn.relu(
        jnp.dot(h, wf1_ref[...], preferred_element_type=jnp.float32)
        + bf1_ref[...])
    out_ref[...] = (
        jnp.dot(h, wf2_ref[...], preferred_element_type=jnp.float32)
        + bf2_ref[...])


def _row_block(i):
    return (i, 0)


_spec_x = pl.BlockSpec((BN, D), _row_block)
_spec_aggp = pl.BlockSpec((NC, BN, D), lambda i: (0, i, 0))
_spec_degp = pl.BlockSpec((NC, BN, D), lambda i: (0, i, 0))
_spec_w = pl.BlockSpec((D, D), lambda i: (0, 0))
_spec_b = pl.BlockSpec((1, D), lambda i: (0, 0))
_spec_wf2 = pl.BlockSpec((D, N_CLS), lambda i: (0, 0))
_spec_bf2 = pl.BlockSpec((1, N_CLS), lambda i: (0, 0))

_GRID = N // BN

_tc_first = pl.pallas_call(
    _tc_first_body,
    grid=(_GRID,),
    in_specs=[_spec_x, _spec_w, _spec_degp],
    out_specs=_spec_x,
    out_shape=jax.ShapeDtypeStruct((N, D), jnp.float32),
)

_tc_mid = pl.pallas_call(
    _tc_mid_body,
    grid=(_GRID,),
    in_specs=[_spec_aggp, _spec_degp, _spec_b, _spec_w],
    out_specs=_spec_x,
    out_shape=jax.ShapeDtypeStruct((N, D), jnp.float32),
)

_tc_final = pl.pallas_call(
    _tc_final_body,
    grid=(_GRID,),
    in_specs=[_spec_aggp, _spec_degp, _spec_b, _spec_w, _spec_b,
              _spec_wf2, _spec_bf2],
    out_specs=pl.BlockSpec((BN, N_CLS), _row_block),
    out_shape=jax.ShapeDtypeStruct((N, N_CLS), jnp.float32),
)


def kernel(x, edge_index, W1, b1, W2, b2, W3, b3, Wf1, bf1, Wf2, bf2):
    row3 = edge_index[0].reshape(NW, NCHUNK, CH)
    col3 = edge_index[1].reshape(NW, NCHUNK, CH)
    rc = jnp.stack([row3, col3], axis=2)
    zeros = jnp.zeros((N, D), jnp.float32)
    ones_ch = jnp.ones((CH, D), jnp.float32)

    degp = _deg_call(col3, ones_ch, zeros)

    t1 = _tc_first(x, W1, degp)
    a1 = _agg_call(t1, rc, zeros)
    t2 = _tc_mid(a1, degp, b1.reshape(1, D), W2)
    a2 = _agg_call(t2, rc, zeros)
    t3 = _tc_mid(a2, degp, b2.reshape(1, D), W3)
    a3 = _agg_call(t3, rc, zeros)
    out = _tc_final(a3, degp, b3.reshape(1, D), Wf1, bf1.reshape(1, D),
                    Wf2, bf2.reshape(1, N_CLS))
    return out
